# Initial kernel scaffold; baseline (speedup 1.0000x reference)
import jax
import jax.numpy as jnp
from jax.experimental import pallas as pl


def kernel(x, edge_index, edge_label_index, W1, b1, W2, b2):
    raise NotImplementedError("write your pallas kernel here")


# speedup vs baseline: 8.6184x; 8.6184x over previous
import functools

import jax
import jax.numpy as jnp
from jax import lax
from jax.experimental import pallas as pl
from jax.experimental.pallas import tpu as pltpu
from jax.experimental.pallas import tpu_sc as plsc

N = 10000
D = 128
E = 320000
L = 200000
NC = 2
NS = 16
NW = NC * NS
RPA = 624
RPB = N - 15 * RPA

EC = 80
EPW = E // NW
ENC = EPW // EC

LC = 128
LNC = 49
LPW = LC * LNC
LPAD = LPW * NW

_F32 = jnp.float32
_I32 = jnp.int32


def _mesh():
    return plsc.VectorSubcoreMesh(
        core_axis_name="c", subcore_axis_name="s",
        num_cores=NC, num_subcores=NS)


def _wid():
    return lax.axis_index("s") * NC + lax.axis_index("c")


def _rows_copy(sid, copy_fn):
    @pl.when(sid != NS - 1)
    def _():
        copy_fn(sid * RPA, RPA)

    @pl.when(sid == NS - 1)
    def _():
        copy_fn((NS - 1) * RPA, RPB)


def _deg_body(dst_hbm, zeros_hbm, ones_hbm, out_hbm, idx_v, ones_v, acc):
    cid = lax.axis_index("c")
    sid = lax.axis_index("s")
    wid = _wid()

    def _init(r0, nr):
        r0 = pl.multiple_of(r0, 8)
        pltpu.sync_copy(zeros_hbm.at[pl.ds(r0, nr)], acc.at[pl.ds(r0, nr)])

    _rows_copy(sid, _init)
    pltpu.sync_copy(ones_hbm, ones_v)
    plsc.subcore_barrier()

    ebase = wid * EPW

    @pl.loop(0, ENC)
    def _chunk(i):
        b = pl.multiple_of(ebase + i * EC, 8)
        pltpu.sync_copy(dst_hbm.at[pl.ds(b, EC)], idx_v)
        pltpu.sync_copy(ones_v, acc.at[idx_v], add=True)

    plsc.subcore_barrier()

    def _wb(r0, nr):
        r0 = pl.multiple_of(r0, 8)
        o0 = pl.multiple_of(cid * N + r0, 8)
        pltpu.sync_copy(acc.at[pl.ds(r0, nr)], out_hbm.at[pl.ds(o0, nr)])

    _rows_copy(sid, _wb)


def _msg_body(hs_hbm, src_hbm, dst_hbm, zeros_hbm, out_hbm,
              sidx_v, didx_v, rows_v, acc, sem):
    cid = lax.axis_index("c")
    sid = lax.axis_index("s")
    wid = _wid()

    def _init(r0, nr):
        r0 = pl.multiple_of(r0, 8)

        @pl.when(cid == 0)
        def _():
            pltpu.sync_copy(hs_hbm.at[pl.ds(r0, nr)], acc.at[pl.ds(r0, nr)])

        @pl.when(cid != 0)
        def _():
            pltpu.sync_copy(zeros_hbm.at[pl.ds(r0, nr)], acc.at[pl.ds(r0, nr)])

    _rows_copy(sid, _init)
    plsc.subcore_barrier()

    ebase = wid * EPW

    @pl.loop(0, ENC)
    def _chunk(i):
        b = pl.multiple_of(ebase + i * EC, 8)
        pltpu.sync_copy(src_hbm.at[pl.ds(b, EC)], sidx_v)
        pltpu.sync_copy(dst_hbm.at[pl.ds(b, EC)], didx_v)
        pltpu.async_copy(hs_hbm.at[sidx_v], rows_v, sem).wait()
        pltpu.sync_copy(rows_v, acc.at[didx_v], add=True)

    plsc.subcore_barrier()

    def _wb(r0, nr):
        r0 = pl.multiple_of(r0, 8)
        o0 = pl.multiple_of(cid * N + r0, 8)
        pltpu.sync_copy(acc.at[pl.ds(r0, nr)], out_hbm.at[pl.ds(o0, nr)])

    _rows_copy(sid, _wb)


def _dec_body(z_hbm, srcl_hbm, dstl_hbm, out_hbm,
              aidx_v, bidx_v, arows_v, brows_v, p16_v, sem):
    wid = _wid()
    base = wid * LPW

    @pl.loop(0, LNC)
    def _chunk(i):
        b = pl.multiple_of(base + i * LC, 8)
        pltpu.sync_copy(srcl_hbm.at[pl.ds(b, LC)], aidx_v)
        pltpu.sync_copy(dstl_hbm.at[pl.ds(b, LC)], bidx_v)
        pltpu.async_copy(z_hbm.at[aidx_v], arows_v, sem).wait()
        pltpu.async_copy(z_hbm.at[bidx_v], brows_v, sem).wait()

        @pl.loop(0, LC)
        def _edge(e):
            p = arows_v[e, pl.ds(0, 16)] * brows_v[e, pl.ds(0, 16)]
            for j in range(1, 8):
                p = p + (arows_v[e, pl.ds(16 * j, 16)]
                         * brows_v[e, pl.ds(16 * j, 16)])
            p16_v[e, :] = p

        pltpu.sync_copy(p16_v, out_hbm.at[pl.ds(b, LC)])


def _ka_body(x_ref, w_ref, d0_ref, d1_ref, o_ref):
    deg = d0_ref[:, 0:1] + d1_ref[:, 0:1] + 1.0
    dinv = lax.rsqrt(deg)
    h = jnp.dot(x_ref[:], w_ref[:], preferred_element_type=_F32)
    o_ref[:] = h * dinv


def _kb_body(a0_ref, a1_ref, d0_ref, d1_ref, b_ref, w_ref, o_ref):
    deg = d0_ref[:, 0:1] + d1_ref[:, 0:1] + 1.0
    dinv = lax.rsqrt(deg)
    h = jnp.maximum((a0_ref[:] + a1_ref[:]) * dinv + b_ref[:], 0.0)
    o_ref[:] = jnp.dot(h, w_ref[:], preferred_element_type=_F32) * dinv


def _kc_body(a0_ref, a1_ref, d0_ref, d1_ref, b_ref, o_ref):
    deg = d0_ref[:, 0:1] + d1_ref[:, 0:1] + 1.0
    dinv = lax.rsqrt(deg)
    o_ref[:] = (a0_ref[:] + a1_ref[:]) * dinv + b_ref[:]


def _kd_body(p_ref, o_ref):
    o_ref[:] = jnp.sum(p_ref[:], axis=1, keepdims=True)


_RB = 2000


def _tc_ka(x, w1, d0, d1):
    return pl.pallas_call(
        _ka_body,
        grid=(N // _RB,),
        in_specs=[
            pl.BlockSpec((_RB, D), lambda i: (i, 0)),
            pl.BlockSpec((D, D), lambda i: (0, 0)),
            pl.BlockSpec((_RB, D), lambda i: (i, 0)),
            pl.BlockSpec((_RB, D), lambda i: (i, 0)),
        ],
        out_specs=pl.BlockSpec((_RB, D), lambda i: (i, 0)),
        out_shape=jax.ShapeDtypeStruct((N, D), _F32),
    )(x, w1, d0, d1)


def _tc_kb(a0, a1, d0, d1, b2d, w2):
    return pl.pallas_call(
        _kb_body,
        grid=(N // _RB,),
        in_specs=[
            pl.BlockSpec((_RB, D), lambda i: (i, 0)),
            pl.BlockSpec((_RB, D), lambda i: (i, 0)),
            pl.BlockSpec((_RB, D), lambda i: (i, 0)),
            pl.BlockSpec((_RB, D), lambda i: (i, 0)),
            pl.BlockSpec((1, D), lambda i: (0, 0)),
            pl.BlockSpec((D, D), lambda i: (0, 0)),
        ],
        out_specs=pl.BlockSpec((_RB, D), lambda i: (i, 0)),
        out_shape=jax.ShapeDtypeStruct((N, D), _F32),
    )(a0, a1, d0, d1, b2d, w2)


def _tc_kc(a0, a1, d0, d1, b2d):
    return pl.pallas_call(
        _kc_body,
        grid=(N // _RB,),
        in_specs=[
            pl.BlockSpec((_RB, D), lambda i: (i, 0)),
            pl.BlockSpec((_RB, D), lambda i: (i, 0)),
            pl.BlockSpec((_RB, D), lambda i: (i, 0)),
            pl.BlockSpec((_RB, D), lambda i: (i, 0)),
            pl.BlockSpec((1, D), lambda i: (0, 0)),
        ],
        out_specs=pl.BlockSpec((_RB, D), lambda i: (i, 0)),
        out_shape=jax.ShapeDtypeStruct((N, D), _F32),
    )(a0, a1, d0, d1, b2d)


_LB = 6272


def _tc_kd(p16):
    return pl.pallas_call(
        _kd_body,
        grid=(LPAD // _LB,),
        in_specs=[pl.BlockSpec((_LB, 16), lambda i: (i, 0))],
        out_specs=pl.BlockSpec((_LB, 1), lambda i: (i, 0)),
        out_shape=jax.ShapeDtypeStruct((LPAD, 1), _F32),
    )(p16)


def kernel(x, edge_index, edge_label_index, W1, b1, W2, b2):
    ei = edge_index.astype(_I32)
    eli = edge_label_index.astype(_I32)
    src, dst = ei[0], ei[1]
    lpad = LPAD - L
    srcl = jnp.concatenate([eli[0], jnp.zeros((lpad,), _I32)])
    dstl = jnp.concatenate([eli[1], jnp.zeros((lpad,), _I32)])

    zeros_nd = jnp.zeros((N, D), _F32)
    ones_ec = jnp.ones((EC, D), _F32)
    b1_2d = b1.reshape(1, D)
    b2_2d = b2.reshape(1, D)

    mesh = _mesh()

    deg_call = pl.kernel(
        _deg_body,
        out_type=jax.ShapeDtypeStruct((2 * N, D), _F32),
        mesh=mesh,
        scratch_types=[
            pltpu.VMEM((EC,), _I32),
            pltpu.VMEM((EC, D), _F32),
            pltpu.MemorySpace.VMEM_SHARED((N, D), _F32),
        ],
    )
    degp = deg_call(dst, zeros_nd, ones_ec)
    d0, d1 = degp[:N], degp[N:]

    msg_call = pl.kernel(
        _msg_body,
        out_type=jax.ShapeDtypeStruct((2 * N, D), _F32),
        mesh=mesh,
        scratch_types=[
            pltpu.VMEM((EC,), _I32),
            pltpu.VMEM((EC,), _I32),
            pltpu.VMEM((EC, D), _F32),
            pltpu.MemorySpace.VMEM_SHARED((N, D), _F32),
            pltpu.SemaphoreType.DMA,
        ],
    )

    hs1 = _tc_ka(x, W1, d0, d1)
    acc1 = msg_call(hs1, src, dst, zeros_nd)
    hs2 = _tc_kb(acc1[:N], acc1[N:], d0, d1, b1_2d, W2)
    acc2 = msg_call(hs2, src, dst, zeros_nd)
    z = _tc_kc(acc2[:N], acc2[N:], d0, d1, b2_2d)

    dec_call = pl.kernel(
        _dec_body,
        out_type=jax.ShapeDtypeStruct((LPAD, 16), _F32),
        mesh=mesh,
        scratch_types=[
            pltpu.VMEM((LC,), _I32),
            pltpu.VMEM((LC,), _I32),
            pltpu.VMEM((LC, D), _F32),
            pltpu.VMEM((LC, D), _F32),
            pltpu.VMEM((LC, 16), _F32),
            pltpu.SemaphoreType.DMA,
        ],
    )
    p16 = dec_call(z, srcl, dstl)
    score = _tc_kd(p16)
    return score[:L, 0]
